# baseline (speedup 1.0000x reference)
import functools

import jax
import jax.numpy as jnp
from jax.experimental import pallas as pl
from jax.experimental.pallas import tpu as pltpu


_LANES = 128
_SUBLANES = 8
_TARGET_TILE_ROWS = 4096


def _mean_matvec_kernel(x_ref, o_ref, *, inv_hw):
    w = jnp.full((x_ref.shape[1], 1), inv_hw, dtype=jnp.float32)
    o_ref[...] = jax.lax.dot_general(
        x_ref[...], w,
        dimension_numbers=(((1,), (0,)), ((), ())),
        preferred_element_type=jnp.float32,
    ).astype(o_ref.dtype)


def _pick_tile_rows(rows):
    tr = min(_TARGET_TILE_ROWS, rows)
    if rows >= 2 * _SUBLANES:
        tr = min(tr, (rows // 2 // _SUBLANES) * _SUBLANES)
        tr = max(tr, _SUBLANES)
    return tr


@functools.partial(jax.jit, static_argnames=())
def kernel(x):
    N, C, H, W = x.shape
    rows = N * C
    hw = H * W
    x2 = x.reshape(rows, hw)
    inv_hw = 1.0 / float(hw)

    tr = _pick_tile_rows(rows)
    num_tiles = pl.cdiv(rows, tr)

    out = pl.pallas_call(
        functools.partial(_mean_matvec_kernel, inv_hw=inv_hw),
        out_shape=jax.ShapeDtypeStruct((rows, 1), x.dtype),
        grid_spec=pl.GridSpec(
            grid=(num_tiles,),
            in_specs=[pl.BlockSpec((tr, hw), lambda i: (i, 0))],
            out_specs=pl.BlockSpec((tr, 1), lambda i: (i, 0)),
        ),
        compiler_params=pltpu.CompilerParams(
            dimension_semantics=("parallel",),
            vmem_limit_bytes=64 * 1024 * 1024,
        ),
        cost_estimate=pl.CostEstimate(
            flops=2 * rows * hw,
            bytes_accessed=rows * hw * x.dtype.itemsize + rows * x.dtype.itemsize,
            transcendentals=0,
        ),
    )(x2)
    return out.reshape(N, C)


# speedup vs baseline: 9.3750x; 9.3750x over previous
import functools

import jax
import jax.numpy as jnp
from jax.experimental import pallas as pl
from jax.experimental.pallas import tpu as pltpu


def _gap_sublane_kernel(x_ref, o_ref, *, inv_hw):
    s = jnp.sum(x_ref[...], axis=1, dtype=jnp.float32)
    o_ref[...] = (s * inv_hw).astype(o_ref.dtype)


@jax.jit
def kernel(x):
    N, C, H, W = x.shape
    hw = H * W
    inv_hw = 1.0 / float(hw)

    y = x.transpose(0, 2, 3, 1).reshape(N, hw, C)

    bn = 8 if N % 8 == 0 else 1
    num_tiles = N // bn

    out = pl.pallas_call(
        functools.partial(_gap_sublane_kernel, inv_hw=inv_hw),
        out_shape=jax.ShapeDtypeStruct((N, C), x.dtype),
        grid_spec=pl.GridSpec(
            grid=(num_tiles,),
            in_specs=[pl.BlockSpec((bn, hw, C), lambda i: (i, 0, 0))],
            out_specs=pl.BlockSpec((bn, C), lambda i: (i, 0)),
        ),
        compiler_params=pltpu.CompilerParams(
            dimension_semantics=("parallel",),
            vmem_limit_bytes=64 * 1024 * 1024,
        ),
        cost_estimate=pl.CostEstimate(
            flops=N * C * H * W,
            bytes_accessed=N * C * H * W * x.dtype.itemsize
            + N * C * x.dtype.itemsize,
            transcendentals=0,
        ),
    )(y)
    return out


# speedup vs baseline: 10.4003x; 1.1094x over previous
import functools

import jax
import jax.numpy as jnp
from jax.experimental import pallas as pl
from jax.experimental.pallas import tpu as pltpu


def _gap_sublane_kernel(x_ref, o_ref, *, inv_hw):
    s = jnp.sum(x_ref[...], axis=1, dtype=jnp.float32)
    o_ref[...] = (s * inv_hw).astype(o_ref.dtype)


@jax.jit
def kernel(x):
    N, C, H, W = x.shape
    hw = H * W
    inv_hw = 1.0 / float(hw)

    y = x.transpose(0, 2, 3, 1).reshape(N, hw, C)

    bn = 16 if N % 16 == 0 else (8 if N % 8 == 0 else 1)
    num_tiles = N // bn

    out = pl.pallas_call(
        functools.partial(_gap_sublane_kernel, inv_hw=inv_hw),
        out_shape=jax.ShapeDtypeStruct((N, C), x.dtype),
        grid_spec=pl.GridSpec(
            grid=(num_tiles,),
            in_specs=[pl.BlockSpec((bn, hw, C), lambda i: (i, 0, 0))],
            out_specs=pl.BlockSpec((bn, C), lambda i: (i, 0)),
        ),
        compiler_params=pltpu.CompilerParams(
            dimension_semantics=("parallel",),
            vmem_limit_bytes=64 * 1024 * 1024,
        ),
        cost_estimate=pl.CostEstimate(
            flops=N * C * H * W,
            bytes_accessed=N * C * H * W * x.dtype.itemsize
            + N * C * x.dtype.itemsize,
            transcendentals=0,
        ),
    )(y)
    return out
